# Initial kernel scaffold; baseline (speedup 1.0000x reference)
import jax
import jax.numpy as jnp
from jax.experimental import pallas as pl


def kernel(x, edge_index, edge_type, basis, root, bias):
    raise NotImplementedError("write your pallas kernel here")


# speedup vs baseline: 22.8026x; 22.8026x over previous
import jax
import jax.numpy as jnp
from jax import lax
from jax.experimental import pallas as pl
from jax.experimental.pallas import tpu as pltpu
from jax.experimental.pallas import tpu_sc as plsc

N, E, D, R = 10000, 320000, 128, 8

NC, NS, L = 2, 16, 16
NW = NC * NS
EW = E // NW
CH = 80
KCH = EW // CH
NP = 10240
ROWS_PER_TILE = NP // NS

BN = 2000


def _xw_body(x_ref, w_ref, y_ref):
    y_ref[0] = jnp.dot(x_ref[...], w_ref[0], preferred_element_type=jnp.float32)


def _transform_table(x, basis):
    return pl.pallas_call(
        _xw_body,
        grid=(R, N // BN),
        in_specs=[
            pl.BlockSpec((BN, D), lambda r, nb: (nb, 0)),
            pl.BlockSpec((1, D, D), lambda r, nb: (r, 0, 0)),
        ],
        out_specs=pl.BlockSpec((1, BN, D), lambda r, nb: (r, nb, 0)),
        out_shape=jax.ShapeDtypeStruct((R, N, D), jnp.float32),
    )(x, basis)


def _sc_body(table, ridx3d, dst3d, slabs,
             acc, idx_all, dst_all, rows, sem):
    c = lax.axis_index("c")
    s = lax.axis_index("s")
    w = c * NS + s

    def _zrow(i, carry):
        for j in range(D // L):
            rows[i, pl.ds(j * L, L)] = jnp.zeros((L,), jnp.float32)
        return carry
    lax.fori_loop(0, CH, _zrow, 0)
    for t in range(ROWS_PER_TILE // CH):
        pltpu.sync_copy(rows, acc.at[pl.ds(s * ROWS_PER_TILE + t * CH, CH)])

    pltpu.sync_copy(ridx3d.at[w], idx_all)
    pltpu.sync_copy(dst3d.at[w], dst_all)

    plsc.subcore_barrier()

    def _chunk(k, carry):
        pltpu.async_copy(table.at[idx_all.at[k]], rows, sem).wait()
        pltpu.sync_copy(rows, acc.at[dst_all.at[k]], add=True)
        return carry
    lax.fori_loop(0, KCH, _chunk, 0)

    plsc.subcore_barrier()

    pltpu.sync_copy(acc.at[pl.ds(s * ROWS_PER_TILE, ROWS_PER_TILE)],
                    slabs.at[c, pl.ds(s * ROWS_PER_TILE, ROWS_PER_TILE)])


def _aggregate(table, ridx3d, dst3d):
    fn = pl.kernel(
        _sc_body,
        out_type=jax.ShapeDtypeStruct((NC, NP, D), jnp.float32),
        mesh=plsc.VectorSubcoreMesh(core_axis_name="c", subcore_axis_name="s"),
        scratch_types=[
            pltpu.VMEM_SHARED((NP, D), jnp.float32),
            pltpu.VMEM((KCH, CH), jnp.int32),
            pltpu.VMEM((KCH, CH), jnp.int32),
            pltpu.VMEM((CH, D), jnp.float32),
            pltpu.SemaphoreType.DMA,
        ],
    )
    return fn(table, ridx3d, dst3d)


def _fin_body(s_ref, x_ref, root_ref, bias_ref, out_ref):
    out_ref[...] = (s_ref[0] + s_ref[1] + bias_ref[...]
                    + jnp.dot(x_ref[...], root_ref[...],
                              preferred_element_type=jnp.float32))


def _finalize(slabs, x, root, bias2d):
    return pl.pallas_call(
        _fin_body,
        grid=(N // BN,),
        in_specs=[
            pl.BlockSpec((NC, BN, D), lambda nb: (0, nb, 0)),
            pl.BlockSpec((BN, D), lambda nb: (nb, 0)),
            pl.BlockSpec((D, D), lambda nb: (0, 0)),
            pl.BlockSpec((1, D), lambda nb: (0, 0)),
        ],
        out_specs=pl.BlockSpec((BN, D), lambda nb: (nb, 0)),
        out_shape=jax.ShapeDtypeStruct((N, D), jnp.float32),
    )(slabs, x, root, bias2d)


def kernel(x, edge_index, edge_type, basis, root, bias):
    src = edge_index[0]
    dst = edge_index[1]
    ridx = edge_type * N + src
    ridx3d = ridx.reshape(NW, KCH, CH)
    dst3d = dst.reshape(NW, KCH, CH)

    table = _transform_table(x, basis).reshape(R * N, D)
    slabs = _aggregate(table, ridx3d, dst3d)
    return _finalize(slabs, x, root, bias.reshape(1, D))
